# baseline (speedup 1.0000x reference)
import functools

import jax
import jax.numpy as jnp
from jax import lax
from jax.experimental import pallas as pl
from jax.experimental.pallas import tpu as pltpu
from jax.experimental.pallas import tpu_sc as plsc

N_NODES = 10000
N_EDGES = 320000
D_FEAT = 128

NC = 2
NS = 16
L = 16
NW = NC * NS

B = 128
C = -(-N_EDGES // (NW * B))
EW = C * B
E_PAD = EW * NW


def _edge_dot_body(h_hbm, src_hbm, dst_hbm, out_hbm,
                   idx_s_v, idx_d_v, rows_s, rows_d, out_v, sem_s, sem_d):
    cid = lax.axis_index("c")
    sid = lax.axis_index("s")
    wid = sid * NC + cid
    base = wid * EW

    pltpu.sync_copy(src_hbm.at[pl.ds(base, EW)], idx_s_v)
    pltpu.sync_copy(dst_hbm.at[pl.ds(base, EW)], idx_d_v)

    lane = lax.iota(jnp.int32, L)

    @pl.loop(0, C)
    def _chunk(c):
        off = c * B
        cps = pltpu.async_copy(h_hbm.at[idx_s_v.at[pl.ds(off, B)]], rows_s,
                               sem_s)
        cpd = pltpu.async_copy(h_hbm.at[idx_d_v.at[pl.ds(off, B)]], rows_d,
                               sem_d)
        cps.wait()
        cpd.wait()
        for g in range(B // L):
            row_ids = lane + (g * L)

            def dbody(d, acc):
                col = jnp.full((L,), d, jnp.int32)
                sv = plsc.load_gather(rows_s, [row_ids, col])
                dv = plsc.load_gather(rows_d, [row_ids, col])
                return acc + sv * dv

            acc = lax.fori_loop(0, D_FEAT, dbody,
                                jnp.zeros((L,), jnp.float32), unroll=8)
            out_v[pl.ds(off + g * L, L)] = acc

    pltpu.sync_copy(out_v, out_hbm.at[pl.ds(base, EW)])


@jax.jit
def kernel(h, edge_index):
    src = edge_index[0].astype(jnp.int32)
    dst = edge_index[1].astype(jnp.int32)
    pad = E_PAD - N_EDGES
    if pad:
        zeros = jnp.zeros((pad,), jnp.int32)
        src = jnp.concatenate([src, zeros])
        dst = jnp.concatenate([dst, zeros])

    mesh = plsc.VectorSubcoreMesh(core_axis_name="c", subcore_axis_name="s",
                                  num_cores=NC, num_subcores=NS)
    run = pl.kernel(
        _edge_dot_body,
        out_type=jax.ShapeDtypeStruct((E_PAD,), jnp.float32),
        mesh=mesh,
        compiler_params=pltpu.CompilerParams(needs_layout_passes=False),
        scratch_types=[
            pltpu.VMEM((EW,), jnp.int32),
            pltpu.VMEM((EW,), jnp.int32),
            pltpu.VMEM((B, D_FEAT), jnp.float32),
            pltpu.VMEM((B, D_FEAT), jnp.float32),
            pltpu.VMEM((EW,), jnp.float32),
            pltpu.SemaphoreType.DMA,
            pltpu.SemaphoreType.DMA,
        ],
    )
    out = run(h, src, dst)
    return out[:N_EDGES]


# speedup vs baseline: 1.7552x; 1.7552x over previous
import jax
import jax.numpy as jnp
from jax import lax
from jax.experimental import pallas as pl
from jax.experimental.pallas import tpu as pltpu
from jax.experimental.pallas import tpu_sc as plsc

N_NODES = 10000
N_EDGES = 320000
D_FEAT = 128

NC = 2
NS = 16
L = 16
NW = NC * NS

B = 128
G = B // L
C = 80
EW = C * B
E_PAD = EW * NW


def _tree_sum(vals):
    while len(vals) > 1:
        vals = [a + b for a, b in zip(vals[::2], vals[1::2])]
    return vals[0]


def _edge_dot_body(h_hbm, src_hbm, dst_hbm, out_hbm,
                   idx_s_v, idx_d_v, rs0, rd0, rs1, rd1, scr, out_v,
                   sem0, sem1):
    cid = lax.axis_index("c")
    sid = lax.axis_index("s")
    wid = sid * NC + cid
    base = wid * EW

    pltpu.sync_copy(src_hbm.at[pl.ds(base, EW)], idx_s_v)
    pltpu.sync_copy(dst_hbm.at[pl.ds(base, EW)], idx_d_v)

    lane = lax.iota(jnp.int32, L)

    def start_gather(c, rs, rd, sem):
        off = c * B
        pltpu.async_copy(h_hbm.at[idx_s_v.at[pl.ds(off, B)]], rs, sem)
        pltpu.async_copy(h_hbm.at[idx_d_v.at[pl.ds(off, B)]], rd, sem)

    def wait_gather(c, rs, rd, sem):
        off = c * B
        pltpu.make_async_copy(h_hbm.at[idx_s_v.at[pl.ds(off, B)]], rs,
                              sem).wait()
        pltpu.make_async_copy(h_hbm.at[idx_d_v.at[pl.ds(off, B)]], rd,
                              sem).wait()

    def compute_chunk(c, rs, rd):
        off = c * B

        @pl.loop(0, G)
        def _group(g):
            eb = g * L
            for i in range(L):
                e = eb + i
                prods = [rs[e, pl.ds(k * L, L)] * rd[e, pl.ds(k * L, L)]
                         for k in range(D_FEAT // L)]
                scr[i, pl.ds(0, L)] = _tree_sum(prods)
            cols = [plsc.load_gather(scr, [lane, jnp.full((L,), j, jnp.int32)])
                    for j in range(L)]
            out_v[pl.ds(off + eb, L)] = _tree_sum(cols)

    start_gather(0, rs0, rd0, sem0)

    @pl.loop(0, C, step=2)
    def _pair(c):
        wait_gather(c, rs0, rd0, sem0)
        start_gather(c + 1, rs1, rd1, sem1)
        compute_chunk(c, rs0, rd0)
        wait_gather(c + 1, rs1, rd1, sem1)

        @pl.when(c + 2 < C)
        def _():
            start_gather(c + 2, rs0, rd0, sem0)

        compute_chunk(c + 1, rs1, rd1)

    pltpu.sync_copy(out_v, out_hbm.at[pl.ds(base, EW)])


@jax.jit
def kernel(h, edge_index):
    src = edge_index[0].astype(jnp.int32)
    dst = edge_index[1].astype(jnp.int32)
    pad = E_PAD - N_EDGES
    if pad:
        zeros = jnp.zeros((pad,), jnp.int32)
        src = jnp.concatenate([src, zeros])
        dst = jnp.concatenate([dst, zeros])

    mesh = plsc.VectorSubcoreMesh(core_axis_name="c", subcore_axis_name="s",
                                  num_cores=NC, num_subcores=NS)
    run = pl.kernel(
        _edge_dot_body,
        out_type=jax.ShapeDtypeStruct((E_PAD,), jnp.float32),
        mesh=mesh,
        compiler_params=pltpu.CompilerParams(needs_layout_passes=False),
        scratch_types=[
            pltpu.VMEM((EW,), jnp.int32),
            pltpu.VMEM((EW,), jnp.int32),
            pltpu.VMEM((B, D_FEAT), jnp.float32),
            pltpu.VMEM((B, D_FEAT), jnp.float32),
            pltpu.VMEM((B, D_FEAT), jnp.float32),
            pltpu.VMEM((B, D_FEAT), jnp.float32),
            pltpu.VMEM((L, L + 1), jnp.float32),
            pltpu.VMEM((EW,), jnp.float32),
            pltpu.SemaphoreType.DMA,
            pltpu.SemaphoreType.DMA,
        ],
    )
    out = run(h, src, dst)
    return out[:N_EDGES]


# speedup vs baseline: 6.2059x; 3.5356x over previous
import jax
import jax.numpy as jnp
from jax import lax
from jax.experimental import pallas as pl
from jax.experimental.pallas import tpu as pltpu
from jax.experimental.pallas import tpu_sc as plsc

N_NODES = 10000
N_EDGES = 320000
D_FEAT = 128
N_PAD = 10112

NC = 2
NS = 16
L = 16
NW = NC * NS

B = 48
G = B // L
C = 210
EW = C * B
E_PAD = EW * NW


def _tree_sum(vals):
    while len(vals) > 1:
        vals = [a + b for a, b in zip(vals[::2], vals[1::2])]
    return vals[0]


def _edge_dot_body(h_hbm, src_hbm, dst_hbm, out_hbm,
                   h_sp, idx_s_v, idx_d_v, rs0, rd0, rs1, rd1, scr, ob0, ob1,
                   sem0, sem1, semo0, semo1):
    cid = lax.axis_index("c")
    sid = lax.axis_index("s")
    wid = sid * NC + cid
    base = wid * EW

    rows_per_tile = N_PAD // NS
    roff = sid * rows_per_tile
    pltpu.sync_copy(h_hbm.at[pl.ds(roff, rows_per_tile)],
                    h_sp.at[pl.ds(roff, rows_per_tile)])

    pltpu.sync_copy(src_hbm.at[pl.ds(base, EW)], idx_s_v)
    pltpu.sync_copy(dst_hbm.at[pl.ds(base, EW)], idx_d_v)
    plsc.subcore_barrier()

    lane = lax.iota(jnp.int32, L)

    def start_gather(c, rs, rd, sem):
        off = c * B
        pltpu.async_copy(h_sp.at[idx_s_v.at[pl.ds(off, B)]], rs, sem)
        pltpu.async_copy(h_sp.at[idx_d_v.at[pl.ds(off, B)]], rd, sem)

    def wait_gather(c, rs, rd, sem):
        off = c * B
        pltpu.make_async_copy(h_sp.at[idx_s_v.at[pl.ds(off, B)]], rs,
                              sem).wait()
        pltpu.make_async_copy(h_sp.at[idx_d_v.at[pl.ds(off, B)]], rd,
                              sem).wait()

    def out_slice(c):
        return out_hbm.at[pl.ds(base + c * B, B)]

    def compute_chunk(c, rs, rd, ob):
        @pl.loop(0, G)
        def _group(g):
            eb = g * L
            for i in range(L):
                e = eb + i
                prods = [rs[e, pl.ds(k * L, L)] * rd[e, pl.ds(k * L, L)]
                         for k in range(D_FEAT // L)]
                scr[i, pl.ds(0, L)] = _tree_sum(prods)
            cols = [plsc.load_gather(scr, [lane, jnp.full((L,), j, jnp.int32)])
                    for j in range(L)]
            ob[pl.ds(eb, L)] = _tree_sum(cols)

    start_gather(0, rs0, rd0, sem0)

    @pl.loop(0, C, step=2)
    def _pair(c):
        wait_gather(c, rs0, rd0, sem0)
        start_gather(c + 1, rs1, rd1, sem1)

        @pl.when(c >= 2)
        def _():
            pltpu.make_async_copy(ob0, out_slice(c - 2), semo0).wait()

        compute_chunk(c, rs0, rd0, ob0)
        pltpu.async_copy(ob0, out_slice(c), semo0)

        wait_gather(c + 1, rs1, rd1, sem1)

        @pl.when(c + 2 < C)
        def _():
            start_gather(c + 2, rs0, rd0, sem0)

        @pl.when(c >= 2)
        def _():
            pltpu.make_async_copy(ob1, out_slice(c - 1), semo1).wait()

        compute_chunk(c + 1, rs1, rd1, ob1)
        pltpu.async_copy(ob1, out_slice(c + 1), semo1)

    pltpu.make_async_copy(ob0, out_slice(C - 2), semo0).wait()
    pltpu.make_async_copy(ob1, out_slice(C - 1), semo1).wait()


@jax.jit
def kernel(h, edge_index):
    h = jnp.pad(h, ((0, N_PAD - N_NODES), (0, 0)))
    src = edge_index[0].astype(jnp.int32)
    dst = edge_index[1].astype(jnp.int32)
    pad = E_PAD - N_EDGES
    if pad:
        zeros = jnp.zeros((pad,), jnp.int32)
        src = jnp.concatenate([src, zeros])
        dst = jnp.concatenate([dst, zeros])

    mesh = plsc.VectorSubcoreMesh(core_axis_name="c", subcore_axis_name="s",
                                  num_cores=NC, num_subcores=NS)
    run = pl.kernel(
        _edge_dot_body,
        out_type=jax.ShapeDtypeStruct((E_PAD,), jnp.float32),
        mesh=mesh,
        compiler_params=pltpu.CompilerParams(needs_layout_passes=False),
        scratch_types=[
            pltpu.VMEM_SHARED((N_PAD, D_FEAT), jnp.float32),
            pltpu.VMEM((EW,), jnp.int32),
            pltpu.VMEM((EW,), jnp.int32),
            pltpu.VMEM((B, D_FEAT), jnp.float32),
            pltpu.VMEM((B, D_FEAT), jnp.float32),
            pltpu.VMEM((B, D_FEAT), jnp.float32),
            pltpu.VMEM((B, D_FEAT), jnp.float32),
            pltpu.VMEM((L, L + 1), jnp.float32),
            pltpu.VMEM((B,), jnp.float32),
            pltpu.VMEM((B,), jnp.float32),
            pltpu.SemaphoreType.DMA,
            pltpu.SemaphoreType.DMA,
            pltpu.SemaphoreType.DMA,
            pltpu.SemaphoreType.DMA,
        ],
    )
    out = run(h, src, dst)
    return out[:N_EDGES]


# speedup vs baseline: 7.7996x; 1.2568x over previous
import jax
import jax.numpy as jnp
from jax import lax
from jax.experimental import pallas as pl
from jax.experimental.pallas import tpu as pltpu
from jax.experimental.pallas import tpu_sc as plsc

N_NODES = 10000
N_EDGES = 320000
D_FEAT = 128
N_PAD = 10240

NC = 2
NS = 16
L = 16
NW = NC * NS

B = 48
G = B // L
PITCH = L + 1
C = 210
EW = C * B
E_PAD = EW * NW


def _tree_sum(vals):
    while len(vals) > 1:
        vals = [a + b for a, b in zip(vals[::2], vals[1::2])]
    return vals[0]


def _edge_dot_body(h_hbm, src_hbm, dst_hbm, out_hbm,
                   h_sp, idx_s_v, idx_d_v, rs0, rd0, rs1, rd1, scr,
                   ob0, ob1, sem0, sem1, semo0, semo1):
    cid = lax.axis_index("c")
    sid = lax.axis_index("s")
    wid = sid * NC + cid
    base = wid * EW

    rows_per_tile = N_PAD // NS
    roff = sid * rows_per_tile
    pltpu.sync_copy(h_hbm.at[pl.ds(roff, rows_per_tile)],
                    h_sp.at[pl.ds(roff, rows_per_tile)])

    pltpu.sync_copy(src_hbm.at[pl.ds(base, EW)], idx_s_v)
    pltpu.sync_copy(dst_hbm.at[pl.ds(base, EW)], idx_d_v)
    plsc.subcore_barrier()

    lane = lax.iota(jnp.int32, L)

    def start_gather(c, rs, rd, sem):
        off = c * B
        pltpu.async_copy(h_sp.at[idx_s_v.at[pl.ds(off, B)]], rs, sem)
        pltpu.async_copy(h_sp.at[idx_d_v.at[pl.ds(off, B)]], rd, sem)

    def wait_gather(c, rs, rd, sem):
        off = c * B
        pltpu.make_async_copy(h_sp.at[idx_s_v.at[pl.ds(off, B)]], rs,
                              sem).wait()
        pltpu.make_async_copy(h_sp.at[idx_d_v.at[pl.ds(off, B)]], rd,
                              sem).wait()

    def start_out(c, ob, sem):
        pltpu.async_copy(ob, out_hbm.at[pl.ds(base + c * B, B)], sem)

    def wait_out(c, ob, sem):
        pltpu.make_async_copy(ob, out_hbm.at[pl.ds(base + c * B, B)],
                              sem).wait()

    def compute_chunk(rs, rd, ob):
        @pl.loop(0, G)
        def _group(g):
            eb = g * L
            for i in range(L):
                e = eb + i
                prods = []
                for k in range(D_FEAT // L):
                    sv = rs[e, pl.ds(k * L, L)]
                    dv = rd[e, pl.ds(k * L, L)]
                    prods.append(sv * dv)
                plsc.store_scatter(scr, [lane + i * PITCH],
                                   _tree_sum(prods))
            cols = [plsc.load_gather(scr, [lane * PITCH + j])
                    for j in range(L)]
            ob[pl.ds(eb, L)] = _tree_sum(cols)

    start_gather(0, rs0, rd0, sem0)

    @pl.loop(0, C, step=2)
    def _pair(c):
        wait_gather(c, rs0, rd0, sem0)
        start_gather(c + 1, rs1, rd1, sem1)

        @pl.when(c >= 2)
        def _():
            wait_out(c - 2, ob0, semo0)

        compute_chunk(rs0, rd0, ob0)
        start_out(c, ob0, semo0)

        wait_gather(c + 1, rs1, rd1, sem1)

        @pl.when(c + 2 < C)
        def _():
            start_gather(c + 2, rs0, rd0, sem0)

        @pl.when(c >= 2)
        def _():
            wait_out(c - 1, ob1, semo1)

        compute_chunk(rs1, rd1, ob1)
        start_out(c + 1, ob1, semo1)

    wait_out(C - 2, ob0, semo0)
    wait_out(C - 1, ob1, semo1)


@jax.jit
def kernel(h, edge_index):
    h = jnp.pad(h, ((0, N_PAD - N_NODES), (0, 0)))
    src = edge_index[0].astype(jnp.int32)
    dst = edge_index[1].astype(jnp.int32)
    pad = E_PAD - N_EDGES
    if pad:
        zeros = jnp.zeros((pad,), jnp.int32)
        src = jnp.concatenate([src, zeros])
        dst = jnp.concatenate([dst, zeros])

    mesh = plsc.VectorSubcoreMesh(core_axis_name="c", subcore_axis_name="s",
                                  num_cores=NC, num_subcores=NS)
    run = pl.kernel(
        _edge_dot_body,
        out_type=jax.ShapeDtypeStruct((E_PAD,), jnp.float32),
        mesh=mesh,
        compiler_params=pltpu.CompilerParams(needs_layout_passes=False),
        scratch_types=[
            pltpu.VMEM_SHARED((N_PAD, D_FEAT), jnp.float32),
            pltpu.VMEM((EW,), jnp.int32),
            pltpu.VMEM((EW,), jnp.int32),
            pltpu.VMEM((B, D_FEAT), jnp.float32),
            pltpu.VMEM((B, D_FEAT), jnp.float32),
            pltpu.VMEM((B, D_FEAT), jnp.float32),
            pltpu.VMEM((B, D_FEAT), jnp.float32),
            pltpu.VMEM((L * PITCH,), jnp.float32),
            pltpu.VMEM((B,), jnp.float32),
            pltpu.VMEM((B,), jnp.float32),
            pltpu.SemaphoreType.DMA,
            pltpu.SemaphoreType.DMA,
            pltpu.SemaphoreType.DMA,
            pltpu.SemaphoreType.DMA,
        ],
    )
    out = run(h, src, dst)
    return out[:N_EDGES]
